# Initial kernel scaffold; baseline (speedup 1.0000x reference)
import jax
import jax.numpy as jnp
from jax.experimental import pallas as pl


def kernel(x, edge_index, edge_attr, W0l, b0l, W0r, b0r, W0e, att0, bias0, W1l, b1l, W1r, b1r, W1e, att1, bias1, W2l, b2l, W2r, b2r, W2e, att2, bias2):
    raise NotImplementedError("write your pallas kernel here")


# speedup vs baseline: 8.6466x; 8.6466x over previous
import functools

import jax
import jax.numpy as jnp
from jax import lax
from jax.experimental import pallas as pl
from jax.experimental.pallas import tpu as pltpu
from jax.experimental.pallas import tpu_sc as plsc

NCORES = 2
NSUB = 16
HID = 32
C = 512


def _leaky(v):
    return jnp.maximum(v, 0.2 * v)


_DNUMS = lax.GatherDimensionNumbers(
    offset_dims=(), collapsed_slice_dims=(0,), start_index_map=(0,))


def _perm(v, idx):
    return lax.gather(v, idx.reshape(16, 1), _DNUMS, (1,),
                      mode=lax.GatherScatterMode.PROMISE_IN_BOUNDS)


def _bcast(v, i):
    return _perm(v, jnp.full((16,), i, jnp.int32))


def _allsum(v):
    lane = lax.iota(jnp.int32, 16)
    for k in (8, 4, 2, 1):
        v = v + _perm(v, lax.bitwise_xor(lane, k))
    return v


def _eproj_tc(ea, We):
    Ep, ed = ea.shape
    BE = 4096

    def body(ea_ref, w_ref, o_ref):
        o_ref[...] = jnp.dot(ea_ref[...], w_ref[...],
                             preferred_element_type=jnp.float32)

    return pl.pallas_call(
        body,
        grid=(Ep // BE,),
        in_specs=[pl.BlockSpec((BE, ed), lambda i: (i, 0)),
                  pl.BlockSpec((ed, HID), lambda i: (0, 0))],
        out_specs=pl.BlockSpec((BE, HID), lambda i: (i, 0)),
        out_shape=jax.ShapeDtypeStruct((Ep, HID), jnp.float32),
    )(ea, We)


def _dense_tc(h_in, acc, bias_prev, la16, Wl, bl, Wr, br, We, att):
    N = h_in.shape[0]
    F = h_in.shape[1] if acc is None else HID
    BN = 2000
    with_epi = acc is not None

    def body(*refs):
        if with_epi:
            (hp_ref, acc_ref, bp_ref, la_ref, wl_ref, bl_ref, wr_ref,
             br_ref, we_ref, att_ref, xl_ref, xre_ref) = refs
            a = acc_ref[...]
            h = (hp_ref[..., :HID] + a[:, :HID]) / (1.0 + a[:, HID:HID + 1] + 1e-16)
            h = jnp.maximum(h + bp_ref[...], 0.0)
        else:
            (hp_ref, la_ref, wl_ref, bl_ref, wr_ref, br_ref, we_ref,
             att_ref, xl_ref, xre_ref) = refs
            h = hp_ref[...]
        la = la_ref[...]
        lattr = la[:, :13] / jnp.maximum(la[:, 13:14], 1.0)
        eL = jnp.dot(lattr, we_ref[...], preferred_element_type=jnp.float32)
        xl = jnp.dot(h, wl_ref[...], preferred_element_type=jnp.float32) + bl_ref[...]
        xr = jnp.dot(h, wr_ref[...], preferred_element_type=jnp.float32) + br_ref[...]
        asf = jnp.sum(_leaky(xl + xr + eL) * att_ref[...], axis=1, keepdims=True)
        xl_ref[...] = jnp.concatenate(
            [xl, jnp.ones((xl.shape[0], 1), jnp.float32),
             jnp.zeros((xl.shape[0], 7), jnp.float32)], axis=1)
        xre_ref[...] = jnp.concatenate(
            [xr, asf, jnp.zeros((xr.shape[0], 15), jnp.float32)], axis=1)

    in_specs = [pl.BlockSpec((BN, h_in.shape[1]), lambda i: (i, 0))]
    args = [h_in]
    if with_epi:
        in_specs += [pl.BlockSpec((BN, 40), lambda i: (i, 0)),
                     pl.BlockSpec((1, HID), lambda i: (0, 0))]
        args += [acc, bias_prev.reshape(1, HID)]
    in_specs += [pl.BlockSpec((BN, 16), lambda i: (i, 0)),
                 pl.BlockSpec((F, HID), lambda i: (0, 0)),
                 pl.BlockSpec((1, HID), lambda i: (0, 0)),
                 pl.BlockSpec((F, HID), lambda i: (0, 0)),
                 pl.BlockSpec((1, HID), lambda i: (0, 0)),
                 pl.BlockSpec((13, HID), lambda i: (0, 0)),
                 pl.BlockSpec((1, HID), lambda i: (0, 0))]
    args += [la16, Wl, bl.reshape(1, HID), Wr, br.reshape(1, HID), We,
             att.reshape(1, HID)]
    return pl.pallas_call(
        body,
        grid=(N // BN,),
        in_specs=in_specs,
        out_specs=[pl.BlockSpec((BN, 40), lambda i: (i, 0)),
                   pl.BlockSpec((BN, 48), lambda i: (i, 0))],
        out_shape=[jax.ShapeDtypeStruct((N, 40), jnp.float32),
                   jax.ShapeDtypeStruct((N, 48), jnp.float32)],
    )(*args)


def _final_tc(xl, acc, bias):
    N = xl.shape[0]
    BN = 2000

    def body(xl_ref, acc_ref, b_ref, o_ref):
        a = acc_ref[...]
        o_ref[...] = (xl_ref[..., :HID] + a[:, :HID]) / (
            1.0 + a[:, HID:HID + 1] + 1e-16) + b_ref[...]

    return pl.pallas_call(
        body,
        grid=(N // BN,),
        in_specs=[pl.BlockSpec((BN, 40), lambda i: (i, 0)),
                  pl.BlockSpec((BN, 40), lambda i: (i, 0)),
                  pl.BlockSpec((1, HID), lambda i: (0, 0))],
        out_specs=pl.BlockSpec((BN, HID), lambda i: (i, 0)),
        out_shape=jax.ShapeDtypeStruct((N, HID), jnp.float32),
    )(xl, acc, bias.reshape(1, HID))


def _sc_mesh():
    return plsc.VectorSubcoreMesh(core_axis_name="c", subcore_axis_name="s",
                                  num_cores=NCORES, num_subcores=NSUB)


def _sc_loop_attr(src, dst, ea1, N, Epad):
    H = N // 2
    Hp = ((H + 128 * NSUB - 1) // (128 * NSUB)) * (128 * NSUB)
    nch = Epad // (NSUB * C)
    rpt = Hp // NSUB
    nz = rpt // 128

    @functools.partial(
        pl.kernel,
        out_type=jax.ShapeDtypeStruct((2 * Hp, 16), jnp.float32),
        mesh=_sc_mesh(),
        compiler_params=pltpu.CompilerParams(use_tc_tiling_on_sc=False),
        scratch_types=[
            pltpu.VMEM((C,), jnp.int32),
            pltpu.VMEM((C,), jnp.int32),
            pltpu.VMEM((C * 13 + 16,), jnp.float32),
            pltpu.VMEM((C, 16), jnp.float32),
            pltpu.VMEM((C // 128, 128), jnp.int32),
            pltpu.VMEM_SHARED((Hp, 16), jnp.float32),
        ],
    )
    def k(src_ref, dst_ref, ea_ref, out_ref, srcb, dstb, eab, rows, idxb, acc):
        cid = lax.axis_index("c")
        sid = lax.axis_index("s")
        base = cid * H
        z16 = jnp.zeros((16,), jnp.float32)

        def zrow(e, _):
            rows[e, pl.ds(0, 16)] = z16
            return 0
        lax.fori_loop(0, C, zrow, 0)
        for z in range(nz):
            pltpu.sync_copy(rows.at[pl.ds(0, 128)],
                            acc.at[pl.ds(sid * rpt + z * 128, 128)])
        plsc.subcore_barrier()

        lane = lax.iota(jnp.int32, 16)

        def chunk(j, _):
            be = (sid * nch + j) * C
            pltpu.sync_copy(src_ref.at[pl.ds(be, C)], srcb)
            pltpu.sync_copy(dst_ref.at[pl.ds(be, C)], dstb)
            pltpu.sync_copy(ea_ref.at[pl.ds(be * 13, C * 13)],
                            eab.at[pl.ds(0, C * 13)])

            def grp(g, _):
                sv16 = srcb[pl.ds(g * 16, 16)]
                dv16 = dstb[pl.ds(g * 16, 16)]
                one = jnp.ones((16,), jnp.float32)
                zero = jnp.zeros((16,), jnp.float32)
                mv16 = (jnp.where(sv16 != dv16, one, zero)
                        * jnp.where(dv16 >= base, one, zero)
                        * jnp.where(dv16 < base + H, one, zero))
                for i in range(16):
                    e = g * 16 + i
                    v = eab[pl.ds(13 * e, 16)]
                    mv = _bcast(mv16, i)
                    r = jnp.where(lane < 13, v * mv,
                                  jnp.where(lane == 13, mv, 0.0))
                    rows[e, pl.ds(0, 16)] = r
                return 0
            lax.fori_loop(0, C // 16, grp, 0)

            for ks in range(C // 128):
                def g16(g, _):
                    dv = dstb[pl.ds(ks * 128 + g * 16, 16)]
                    idxb[ks, pl.ds(g * 16, 16)] = jnp.clip(dv - base, 0, H - 1)
                    return 0
                lax.fori_loop(0, 8, g16, 0)
            for ks in range(C // 128):
                pltpu.sync_copy(rows.at[pl.ds(ks * 128, 128)],
                                acc.at[idxb.at[ks]], add=True)
            return 0
        lax.fori_loop(0, nch, chunk, 0)
        plsc.subcore_barrier()
        pltpu.sync_copy(acc.at[pl.ds(sid * rpt, rpt)],
                        out_ref.at[pl.ds(cid * Hp + sid * rpt, rpt)])

    out = k(src, dst, ea1)
    return jnp.concatenate([out[:H], out[Hp:Hp + H]], axis=0)


def _sc_alpha_pass(src, dst, xl, xre, eproj, att, N, Epad):
    NW = NCORES * NSUB
    nch = Epad // (NW * C)

    @functools.partial(
        pl.kernel,
        out_type=jax.ShapeDtypeStruct((Epad,), jnp.float32),
        mesh=_sc_mesh(),
        compiler_params=pltpu.CompilerParams(use_tc_tiling_on_sc=False),
        scratch_types=[
            pltpu.VMEM((C,), jnp.int32),
            pltpu.VMEM((C,), jnp.int32),
            pltpu.VMEM((C, 40), jnp.float32),
            pltpu.VMEM((C, 48), jnp.float32),
            pltpu.VMEM((C, HID), jnp.float32),
            pltpu.VMEM((HID,), jnp.float32),
            pltpu.VMEM((C,), jnp.float32),
            pltpu.VMEM((C,), jnp.float32),
            pltpu.SemaphoreType.DMA,
            pltpu.SemaphoreType.DMA,
        ],
    )
    def k(src_ref, dst_ref, xl_ref, xre_ref, ep_ref, att_ref, out_ref,
          srcb, dstb, xlb, xrb, epb, attb, ab, exb, sem, sem2):
        cid = lax.axis_index("c")
        sid = lax.axis_index("s")
        w = cid * NSUB + sid
        z16 = jnp.zeros((16,), jnp.float32)
        lane = lax.iota(jnp.int32, 16)
        pltpu.sync_copy(att_ref, attb)

        def chunk(j, _):
            be = (w * nch + j) * C
            pltpu.sync_copy(src_ref.at[pl.ds(be, C)], srcb)
            pltpu.sync_copy(dst_ref.at[pl.ds(be, C)], dstb)
            pltpu.sync_copy(ep_ref.at[pl.ds(be, C)], epb)
            cps = []
            for ks in range(C // 128):
                cps.append(pltpu.async_copy(
                    xl_ref.at[srcb.at[pl.ds(ks * 128, 128)]],
                    xlb.at[pl.ds(ks * 128, 128)], sem))
                cps.append(pltpu.async_copy(
                    xre_ref.at[dstb.at[pl.ds(ks * 128, 128)]],
                    xrb.at[pl.ds(ks * 128, 128)], sem2))
            for cp in cps:
                cp.wait()

            at0 = attb[pl.ds(0, 16)]
            at1 = attb[pl.ds(16, 16)]

            def grpA(g, _):
                avec = z16
                for i in range(16):
                    e = g * 16 + i
                    v0 = xlb[e, pl.ds(0, 16)] + xrb[e, pl.ds(0, 16)] + epb[e, pl.ds(0, 16)]
                    v1 = xlb[e, pl.ds(16, 16)] + xrb[e, pl.ds(16, 16)] + epb[e, pl.ds(16, 16)]
                    sv = _leaky(v0) * at0 + _leaky(v1) * at1
                    a_i = _allsum(sv) - _bcast(xrb[e, pl.ds(32, 16)], 0)
                    avec = jnp.where(lane == i, a_i, avec)
                ab[pl.ds(g * 16, 16)] = avec
                return 0
            lax.fori_loop(0, C // 16, grpA, 0)

            one = jnp.ones((16,), jnp.float32)
            zero = jnp.zeros((16,), jnp.float32)

            def g16(g, _):
                gg = g * 16
                dv = dstb[pl.ds(gg, 16)]
                sv2 = srcb[pl.ds(gg, 16)]
                av = ab[pl.ds(gg, 16)]
                mf = jnp.where(sv2 != dv, one, zero)
                exb[pl.ds(gg, 16)] = jnp.exp(av) * mf
                return 0
            lax.fori_loop(0, C // 16, g16, 0)
            pltpu.sync_copy(exb, out_ref.at[pl.ds(be, C)])
            return 0
        lax.fori_loop(0, nch, chunk, 0)

    return k(src, dst, xl, xre, eproj, att)


def _sc_scatter_pass(src, dst, ex, xl, N, Epad):
    H = N // 2
    Hp = ((H + 127) // 128) * 128
    C2 = 128
    nch = Epad // (NSUB * C2)
    rpt = Hp // NSUB
    nzf = rpt // 128
    nzr = rpt - nzf * 128

    @functools.partial(
        pl.kernel,
        out_type=jax.ShapeDtypeStruct((2 * Hp, 40), jnp.float32),
        mesh=_sc_mesh(),
        compiler_params=pltpu.CompilerParams(use_tc_tiling_on_sc=False),
        scratch_types=[
            pltpu.VMEM((C2,), jnp.int32),
            pltpu.VMEM((C2,), jnp.int32),
            pltpu.VMEM((C2,), jnp.float32),
            pltpu.VMEM((C2, 40), jnp.float32),
            pltpu.VMEM((C2,), jnp.int32),
            pltpu.VMEM_SHARED((Hp, 40), jnp.float32),
            pltpu.SemaphoreType.DMA,
        ],
    )
    def k(src_ref, dst_ref, ex_ref, xl_ref, out_ref,
          srcb, dstb, exc, xlb, idxb, acc, sem):
        cid = lax.axis_index("c")
        sid = lax.axis_index("s")
        base = cid * H
        z16 = jnp.zeros((16,), jnp.float32)
        lane = lax.iota(jnp.int32, 16)

        def zrow(e, _):
            xlb[e, pl.ds(0, 16)] = z16
            xlb[e, pl.ds(16, 16)] = z16
            xlb[e, pl.ds(24, 16)] = z16
            return 0
        lax.fori_loop(0, C2, zrow, 0)
        for z in range(nzf):
            pltpu.sync_copy(xlb.at[pl.ds(0, 128)],
                            acc.at[pl.ds(sid * rpt + z * 128, 128)])
        if nzr:
            pltpu.sync_copy(xlb.at[pl.ds(0, nzr)],
                            acc.at[pl.ds(sid * rpt + nzf * 128, nzr)])
        plsc.subcore_barrier()

        one = jnp.ones((16,), jnp.float32)
        zero = jnp.zeros((16,), jnp.float32)

        def chunk(j, _):
            be = (sid * nch + j) * C2
            pltpu.sync_copy(src_ref.at[pl.ds(be, C2)], srcb)
            pltpu.sync_copy(dst_ref.at[pl.ds(be, C2)], dstb)
            pltpu.sync_copy(ex_ref.at[pl.ds(be, C2)], exc)
            pltpu.async_copy(xl_ref.at[srcb], xlb, sem).wait()

            def g16(g, _):
                gg = g * 16
                dv = dstb[pl.ds(gg, 16)]
                mh = (jnp.where(dv >= base, one, zero)
                      * jnp.where(dv < base + H, one, zero))
                exc[pl.ds(gg, 16)] = exc[pl.ds(gg, 16)] * mh
                idxb[pl.ds(gg, 16)] = jnp.clip(dv - base, 0, H - 1)
                return 0
            lax.fori_loop(0, C2 // 16, g16, 0)

            def grpC(g, _):
                exv16 = exc[pl.ds(g * 16, 16)]
                for i in range(16):
                    e = g * 16 + i
                    exv = _bcast(exv16, i)
                    l0 = xlb[e, pl.ds(0, 16)]
                    l1 = xlb[e, pl.ds(16, 16)]
                    l2 = xlb[e, pl.ds(24, 16)]
                    xlb[e, pl.ds(0, 16)] = l0 * exv
                    xlb[e, pl.ds(16, 16)] = l1 * exv
                    xlb[e, pl.ds(24, 16)] = l2 * exv
                return 0
            lax.fori_loop(0, C2 // 16, grpC, 0)

            pltpu.sync_copy(xlb, acc.at[idxb], add=True)
            return 0
        lax.fori_loop(0, nch, chunk, 0)
        plsc.subcore_barrier()
        pltpu.sync_copy(acc.at[pl.ds(sid * rpt, rpt)],
                        out_ref.at[pl.ds(cid * Hp + sid * rpt, rpt)])

    out = k(src, dst, ex, xl)
    return jnp.concatenate([out[:H], out[Hp:Hp + H]], axis=0)


def kernel(x, edge_index, edge_attr,
           W0l, b0l, W0r, b0r, W0e, att0, bias0,
           W1l, b1l, W1r, b1r, W1e, att1, bias1,
           W2l, b2l, W2r, b2r, W2e, att2, bias2):
    N = x.shape[0]
    E = edge_index.shape[1]
    ed = edge_attr.shape[1]
    chw = NCORES * NSUB * C
    Epad = ((E + chw - 1) // chw) * chw
    pad = Epad - E

    src = jnp.concatenate([edge_index[0], jnp.zeros((pad,), jnp.int32)])
    dst = jnp.concatenate([edge_index[1], jnp.zeros((pad,), jnp.int32)])
    ea = jnp.concatenate([edge_attr, jnp.zeros((pad, ed), jnp.float32)], axis=0)
    ea1 = ea.reshape(-1)

    la16 = _sc_loop_attr(src, dst, ea1, N, Epad)

    params = [
        (W0l, b0l, W0r, b0r, W0e, att0, bias0),
        (W1l, b1l, W1r, b1r, W1e, att1, bias1),
        (W2l, b2l, W2r, b2r, W2e, att2, bias2),
    ]

    h_in = x
    acc = None
    bias_prev = None
    xl = None
    for l in range(3):
        Wl, bl, Wr, br, We, att, bias = params[l]
        eproj = _eproj_tc(ea, We)
        xl, xre = _dense_tc(h_in, acc, bias_prev, la16, Wl, bl, Wr, br, We, att)
        ex = _sc_alpha_pass(src, dst, xl, xre, eproj, att, N, Epad)
        acc = _sc_scatter_pass(src, dst, ex, xl, N, Epad)
        h_in = xl
        bias_prev = bias
    return _final_tc(xl, acc, bias2)


# speedup vs baseline: 10.6119x; 1.2273x over previous
import functools

import jax
import jax.numpy as jnp
from jax import lax
from jax.experimental import pallas as pl
from jax.experimental.pallas import tpu as pltpu
from jax.experimental.pallas import tpu_sc as plsc

NCORES = 2
NSUB = 16
HID = 32
C = 512


def _leaky(v):
    return jnp.maximum(v, 0.2 * v)


_DNUMS = lax.GatherDimensionNumbers(
    offset_dims=(), collapsed_slice_dims=(0,), start_index_map=(0,))


def _perm(v, idx):
    return lax.gather(v, idx.reshape(16, 1), _DNUMS, (1,),
                      mode=lax.GatherScatterMode.PROMISE_IN_BOUNDS)


def _bcast(v, i):
    return _perm(v, jnp.full((16,), i, jnp.int32))


def _allsum(v):
    lane = lax.iota(jnp.int32, 16)
    for k in (8, 4, 2, 1):
        v = v + _perm(v, lax.bitwise_xor(lane, k))
    return v


def _eproj_tc(ea, We):
    Ep, ed = ea.shape
    BE = 4096

    def body(ea_ref, w_ref, o_ref):
        o_ref[...] = jnp.dot(ea_ref[...], w_ref[...],
                             preferred_element_type=jnp.float32)

    return pl.pallas_call(
        body,
        grid=(Ep // BE,),
        in_specs=[pl.BlockSpec((BE, ed), lambda i: (i, 0)),
                  pl.BlockSpec((ed, HID), lambda i: (0, 0))],
        out_specs=pl.BlockSpec((BE, HID), lambda i: (i, 0)),
        out_shape=jax.ShapeDtypeStruct((Ep, HID), jnp.float32),
    )(ea, We)


def _dense_tc(h_in, acc, bias_prev, la16, Wl, bl, Wr, br, We, att):
    N = h_in.shape[0]
    F = h_in.shape[1] if acc is None else HID
    BN = 2000
    with_epi = acc is not None

    def body(*refs):
        if with_epi:
            (hp_ref, acc_ref, bp_ref, la_ref, wl_ref, bl_ref, wr_ref,
             br_ref, we_ref, att_ref, xl_ref, xre_ref) = refs
            a = acc_ref[...]
            h = (hp_ref[..., :HID] + a[:, :HID]) / (1.0 + a[:, HID:HID + 1] + 1e-16)
            h = jnp.maximum(h + bp_ref[...], 0.0)
        else:
            (hp_ref, la_ref, wl_ref, bl_ref, wr_ref, br_ref, we_ref,
             att_ref, xl_ref, xre_ref) = refs
            h = hp_ref[...]
        la = la_ref[...]
        lattr = la[:, :13] / jnp.maximum(la[:, 13:14], 1.0)
        eL = jnp.dot(lattr, we_ref[...], preferred_element_type=jnp.float32)
        xl = jnp.dot(h, wl_ref[...], preferred_element_type=jnp.float32) + bl_ref[...]
        xr = jnp.dot(h, wr_ref[...], preferred_element_type=jnp.float32) + br_ref[...]
        asf = jnp.sum(_leaky(xl + xr + eL) * att_ref[...], axis=1, keepdims=True)
        xl_ref[...] = jnp.concatenate(
            [xl, jnp.ones((xl.shape[0], 1), jnp.float32),
             jnp.zeros((xl.shape[0], 7), jnp.float32)], axis=1)
        xre_ref[...] = jnp.concatenate(
            [xr, asf, jnp.zeros((xr.shape[0], 15), jnp.float32)], axis=1)

    in_specs = [pl.BlockSpec((BN, h_in.shape[1]), lambda i: (i, 0))]
    args = [h_in]
    if with_epi:
        in_specs += [pl.BlockSpec((BN, 40), lambda i: (i, 0)),
                     pl.BlockSpec((1, HID), lambda i: (0, 0))]
        args += [acc, bias_prev.reshape(1, HID)]
    in_specs += [pl.BlockSpec((BN, 16), lambda i: (i, 0)),
                 pl.BlockSpec((F, HID), lambda i: (0, 0)),
                 pl.BlockSpec((1, HID), lambda i: (0, 0)),
                 pl.BlockSpec((F, HID), lambda i: (0, 0)),
                 pl.BlockSpec((1, HID), lambda i: (0, 0)),
                 pl.BlockSpec((13, HID), lambda i: (0, 0)),
                 pl.BlockSpec((1, HID), lambda i: (0, 0))]
    args += [la16, Wl, bl.reshape(1, HID), Wr, br.reshape(1, HID), We,
             att.reshape(1, HID)]
    return pl.pallas_call(
        body,
        grid=(N // BN,),
        in_specs=in_specs,
        out_specs=[pl.BlockSpec((BN, 40), lambda i: (i, 0)),
                   pl.BlockSpec((BN, 48), lambda i: (i, 0))],
        out_shape=[jax.ShapeDtypeStruct((N, 40), jnp.float32),
                   jax.ShapeDtypeStruct((N, 48), jnp.float32)],
    )(*args)


def _final_tc(xl, acc, bias):
    N = xl.shape[0]
    BN = 2000

    def body(xl_ref, acc_ref, b_ref, o_ref):
        a = acc_ref[...]
        o_ref[...] = (xl_ref[..., :HID] + a[:, :HID]) / (
            1.0 + a[:, HID:HID + 1] + 1e-16) + b_ref[...]

    return pl.pallas_call(
        body,
        grid=(N // BN,),
        in_specs=[pl.BlockSpec((BN, 40), lambda i: (i, 0)),
                  pl.BlockSpec((BN, 40), lambda i: (i, 0)),
                  pl.BlockSpec((1, HID), lambda i: (0, 0))],
        out_specs=pl.BlockSpec((BN, HID), lambda i: (i, 0)),
        out_shape=jax.ShapeDtypeStruct((N, HID), jnp.float32),
    )(xl, acc, bias.reshape(1, HID))


def _sc_mesh():
    return plsc.VectorSubcoreMesh(core_axis_name="c", subcore_axis_name="s",
                                  num_cores=NCORES, num_subcores=NSUB)


def _sc_loop_attr(src, dst, ea1, N, Epad):
    H = N // 2
    Hp = ((H + 128 * NSUB - 1) // (128 * NSUB)) * (128 * NSUB)
    nch = Epad // (NSUB * C)
    rpt = Hp // NSUB
    nz = rpt // 128

    @functools.partial(
        pl.kernel,
        out_type=jax.ShapeDtypeStruct((2 * Hp, 16), jnp.float32),
        mesh=_sc_mesh(),
        compiler_params=pltpu.CompilerParams(use_tc_tiling_on_sc=False),
        scratch_types=[
            pltpu.VMEM((C,), jnp.int32),
            pltpu.VMEM((C,), jnp.int32),
            pltpu.VMEM((C * 13 + 16,), jnp.float32),
            pltpu.VMEM((C, 16), jnp.float32),
            pltpu.VMEM((C // 128, 128), jnp.int32),
            pltpu.VMEM_SHARED((Hp, 16), jnp.float32),
        ],
    )
    def k(src_ref, dst_ref, ea_ref, out_ref, srcb, dstb, eab, rows, idxb, acc):
        cid = lax.axis_index("c")
        sid = lax.axis_index("s")
        base = cid * H
        z16 = jnp.zeros((16,), jnp.float32)

        def zrow(e, _):
            rows[e, pl.ds(0, 16)] = z16
            return 0
        lax.fori_loop(0, C, zrow, 0)
        for z in range(nz):
            pltpu.sync_copy(rows.at[pl.ds(0, 128)],
                            acc.at[pl.ds(sid * rpt + z * 128, 128)])
        plsc.subcore_barrier()

        lane = lax.iota(jnp.int32, 16)

        def chunk(j, _):
            be = (sid * nch + j) * C
            pltpu.sync_copy(src_ref.at[pl.ds(be, C)], srcb)
            pltpu.sync_copy(dst_ref.at[pl.ds(be, C)], dstb)
            pltpu.sync_copy(ea_ref.at[pl.ds(be * 13, C * 13)],
                            eab.at[pl.ds(0, C * 13)])

            def grp(g, _):
                sv16 = srcb[pl.ds(g * 16, 16)]
                dv16 = dstb[pl.ds(g * 16, 16)]
                one = jnp.ones((16,), jnp.float32)
                zero = jnp.zeros((16,), jnp.float32)
                mv16 = (jnp.where(sv16 != dv16, one, zero)
                        * jnp.where(dv16 >= base, one, zero)
                        * jnp.where(dv16 < base + H, one, zero))
                for i in range(16):
                    e = g * 16 + i
                    v = eab[pl.ds(13 * e, 16)]
                    mv = _bcast(mv16, i)
                    r = jnp.where(lane < 13, v * mv,
                                  jnp.where(lane == 13, mv, 0.0))
                    rows[e, pl.ds(0, 16)] = r
                return 0
            lax.fori_loop(0, C // 16, grp, 0)

            for ks in range(C // 128):
                def g16(g, _):
                    dv = dstb[pl.ds(ks * 128 + g * 16, 16)]
                    idxb[ks, pl.ds(g * 16, 16)] = jnp.clip(dv - base, 0, H - 1)
                    return 0
                lax.fori_loop(0, 8, g16, 0)
            for ks in range(C // 128):
                pltpu.sync_copy(rows.at[pl.ds(ks * 128, 128)],
                                acc.at[idxb.at[ks]], add=True)
            return 0
        lax.fori_loop(0, nch, chunk, 0)
        plsc.subcore_barrier()
        pltpu.sync_copy(acc.at[pl.ds(sid * rpt, rpt)],
                        out_ref.at[pl.ds(cid * Hp + sid * rpt, rpt)])

    out = k(src, dst, ea1)
    return jnp.concatenate([out[:H], out[Hp:Hp + H]], axis=0)


def _sc_alpha_pass(src, dst, xl, xre, eproj, att, N, Epad):
    NW = NCORES * NSUB
    nch = Epad // (NW * C)

    @functools.partial(
        pl.kernel,
        out_type=jax.ShapeDtypeStruct((Epad,), jnp.float32),
        mesh=_sc_mesh(),
        compiler_params=pltpu.CompilerParams(use_tc_tiling_on_sc=False),
        scratch_types=[
            pltpu.VMEM((C,), jnp.int32),
            pltpu.VMEM((C,), jnp.int32),
            pltpu.VMEM((C, 40), jnp.float32),
            pltpu.VMEM((C, 48), jnp.float32),
            pltpu.VMEM((C, HID), jnp.float32),
            pltpu.VMEM((HID,), jnp.float32),
            pltpu.VMEM((C,), jnp.float32),
            pltpu.VMEM((C,), jnp.float32),
            pltpu.SemaphoreType.DMA,
            pltpu.SemaphoreType.DMA,
        ],
    )
    def k(src_ref, dst_ref, xl_ref, xre_ref, ep_ref, att_ref, out_ref,
          srcb, dstb, xlb, xrb, epb, attb, ab, exb, sem, sem2):
        cid = lax.axis_index("c")
        sid = lax.axis_index("s")
        w = cid * NSUB + sid
        z16 = jnp.zeros((16,), jnp.float32)
        lane = lax.iota(jnp.int32, 16)
        pltpu.sync_copy(att_ref, attb)

        def chunk(j, _):
            be = (w * nch + j) * C
            c1 = pltpu.async_copy(src_ref.at[pl.ds(be, C)], srcb, sem)
            c2 = pltpu.async_copy(dst_ref.at[pl.ds(be, C)], dstb, sem)
            c3 = pltpu.async_copy(ep_ref.at[pl.ds(be, C)], epb, sem2)
            c1.wait()
            c2.wait()
            cps = []
            for ks in range(C // 128):
                cps.append(pltpu.async_copy(
                    xl_ref.at[srcb.at[pl.ds(ks * 128, 128)]],
                    xlb.at[pl.ds(ks * 128, 128)], sem))
                cps.append(pltpu.async_copy(
                    xre_ref.at[dstb.at[pl.ds(ks * 128, 128)]],
                    xrb.at[pl.ds(ks * 128, 128)], sem2))
            c3.wait()
            for cp in cps:
                cp.wait()

            at0 = attb[pl.ds(0, 16)]
            at1 = attb[pl.ds(16, 16)]

            def grpA(g, _):
                avec = z16
                for i in range(16):
                    e = g * 16 + i
                    v0 = xlb[e, pl.ds(0, 16)] + xrb[e, pl.ds(0, 16)] + epb[e, pl.ds(0, 16)]
                    v1 = xlb[e, pl.ds(16, 16)] + xrb[e, pl.ds(16, 16)] + epb[e, pl.ds(16, 16)]
                    sv = _leaky(v0) * at0 + _leaky(v1) * at1
                    a_i = _allsum(sv) - _bcast(xrb[e, pl.ds(32, 16)], 0)
                    avec = jnp.where(lane == i, a_i, avec)
                ab[pl.ds(g * 16, 16)] = avec
                return 0
            lax.fori_loop(0, C // 16, grpA, 0)

            one = jnp.ones((16,), jnp.float32)
            zero = jnp.zeros((16,), jnp.float32)

            def g16(g, _):
                gg = g * 16
                dv = dstb[pl.ds(gg, 16)]
                sv2 = srcb[pl.ds(gg, 16)]
                av = ab[pl.ds(gg, 16)]
                mf = jnp.where(sv2 != dv, one, zero)
                exb[pl.ds(gg, 16)] = jnp.exp(av) * mf
                return 0
            lax.fori_loop(0, C // 16, g16, 0)
            pltpu.sync_copy(exb, out_ref.at[pl.ds(be, C)])
            return 0
        lax.fori_loop(0, nch, chunk, 0)

    return k(src, dst, xl, xre, eproj, att)


def _sc_scatter_pass(src, dst, ex, xl, N, Epad):
    H = N // 2
    Hp = ((H + 127) // 128) * 128
    C2 = 128
    nch = Epad // (NSUB * C2)
    rpt = Hp // NSUB
    nzf = rpt // 128
    nzr = rpt - nzf * 128

    @functools.partial(
        pl.kernel,
        out_type=jax.ShapeDtypeStruct((2 * Hp, 40), jnp.float32),
        mesh=_sc_mesh(),
        compiler_params=pltpu.CompilerParams(use_tc_tiling_on_sc=False),
        scratch_types=[
            pltpu.VMEM((C2,), jnp.int32),
            pltpu.VMEM((C2,), jnp.int32),
            pltpu.VMEM((C2,), jnp.float32),
            pltpu.VMEM((C2, 40), jnp.float32),
            pltpu.VMEM((C2,), jnp.int32),
            pltpu.VMEM_SHARED((Hp, 40), jnp.float32),
            pltpu.SemaphoreType.DMA,
            pltpu.SemaphoreType.DMA,
        ],
    )
    def k(src_ref, dst_ref, ex_ref, xl_ref, out_ref,
          srcb, dstb, exc, xlb, idxb, acc, sem, sem2):
        cid = lax.axis_index("c")
        sid = lax.axis_index("s")
        base = cid * H
        z16 = jnp.zeros((16,), jnp.float32)
        lane = lax.iota(jnp.int32, 16)

        def zrow(e, _):
            xlb[e, pl.ds(0, 16)] = z16
            xlb[e, pl.ds(16, 16)] = z16
            xlb[e, pl.ds(24, 16)] = z16
            return 0
        lax.fori_loop(0, C2, zrow, 0)
        for z in range(nzf):
            pltpu.sync_copy(xlb.at[pl.ds(0, 128)],
                            acc.at[pl.ds(sid * rpt + z * 128, 128)])
        if nzr:
            pltpu.sync_copy(xlb.at[pl.ds(0, nzr)],
                            acc.at[pl.ds(sid * rpt + nzf * 128, nzr)])
        plsc.subcore_barrier()

        one = jnp.ones((16,), jnp.float32)
        zero = jnp.zeros((16,), jnp.float32)

        def chunk(j, _):
            be = (sid * nch + j) * C2
            pltpu.sync_copy(src_ref.at[pl.ds(be, C2)], srcb)
            g = pltpu.async_copy(xl_ref.at[srcb], xlb, sem)
            c1 = pltpu.async_copy(dst_ref.at[pl.ds(be, C2)], dstb, sem2)
            c2 = pltpu.async_copy(ex_ref.at[pl.ds(be, C2)], exc, sem2)
            c1.wait()
            c2.wait()
            g.wait()

            def g16(g, _):
                gg = g * 16
                dv = dstb[pl.ds(gg, 16)]
                mh = (jnp.where(dv >= base, one, zero)
                      * jnp.where(dv < base + H, one, zero))
                exc[pl.ds(gg, 16)] = exc[pl.ds(gg, 16)] * mh
                idxb[pl.ds(gg, 16)] = jnp.clip(dv - base, 0, H - 1)
                return 0
            lax.fori_loop(0, C2 // 16, g16, 0)

            def grpC(g, _):
                exv16 = exc[pl.ds(g * 16, 16)]
                for i in range(16):
                    e = g * 16 + i
                    exv = _bcast(exv16, i)
                    l0 = xlb[e, pl.ds(0, 16)]
                    l1 = xlb[e, pl.ds(16, 16)]
                    l2 = xlb[e, pl.ds(24, 16)]
                    xlb[e, pl.ds(0, 16)] = l0 * exv
                    xlb[e, pl.ds(16, 16)] = l1 * exv
                    xlb[e, pl.ds(24, 16)] = l2 * exv
                return 0
            lax.fori_loop(0, C2 // 16, grpC, 0)

            pltpu.sync_copy(xlb, acc.at[idxb], add=True)
            return 0
        lax.fori_loop(0, nch, chunk, 0)
        plsc.subcore_barrier()
        pltpu.sync_copy(acc.at[pl.ds(sid * rpt, rpt)],
                        out_ref.at[pl.ds(cid * Hp + sid * rpt, rpt)])

    out = k(src, dst, ex, xl)
    return jnp.concatenate([out[:H], out[Hp:Hp + H]], axis=0)


def kernel(x, edge_index, edge_attr,
           W0l, b0l, W0r, b0r, W0e, att0, bias0,
           W1l, b1l, W1r, b1r, W1e, att1, bias1,
           W2l, b2l, W2r, b2r, W2e, att2, bias2):
    N = x.shape[0]
    E = edge_index.shape[1]
    ed = edge_attr.shape[1]
    chw = NCORES * NSUB * C
    Epad = ((E + chw - 1) // chw) * chw
    pad = Epad - E

    src = jnp.concatenate([edge_index[0], jnp.zeros((pad,), jnp.int32)])
    dst = jnp.concatenate([edge_index[1], jnp.zeros((pad,), jnp.int32)])
    ea = jnp.concatenate([edge_attr, jnp.zeros((pad, ed), jnp.float32)], axis=0)
    ea1 = ea.reshape(-1)

    la16 = _sc_loop_attr(src, dst, ea1, N, Epad)

    params = [
        (W0l, b0l, W0r, b0r, W0e, att0, bias0),
        (W1l, b1l, W1r, b1r, W1e, att1, bias1),
        (W2l, b2l, W2r, b2r, W2e, att2, bias2),
    ]

    h_in = x
    acc = None
    bias_prev = None
    xl = None
    for l in range(3):
        Wl, bl, Wr, br, We, att, bias = params[l]
        eproj = _eproj_tc(ea, We)
        xl, xre = _dense_tc(h_in, acc, bias_prev, la16, Wl, bl, Wr, br, We, att)
        ex = _sc_alpha_pass(src, dst, xl, xre, eproj, att, N, Epad)
        acc = _sc_scatter_pass(src, dst, ex, xl, N, Epad)
        h_in = xl
        bias_prev = bias
    return _final_tc(xl, acc, bias2)


# speedup vs baseline: 10.7250x; 1.0107x over previous
import functools

import jax
import jax.numpy as jnp
from jax import lax
from jax.experimental import pallas as pl
from jax.experimental.pallas import tpu as pltpu
from jax.experimental.pallas import tpu_sc as plsc

NCORES = 2
NSUB = 16
HID = 32
C = 1024


def _leaky(v):
    return jnp.maximum(v, 0.2 * v)


_DNUMS = lax.GatherDimensionNumbers(
    offset_dims=(), collapsed_slice_dims=(0,), start_index_map=(0,))


def _perm(v, idx):
    return lax.gather(v, idx.reshape(16, 1), _DNUMS, (1,),
                      mode=lax.GatherScatterMode.PROMISE_IN_BOUNDS)


def _bcast(v, i):
    return _perm(v, jnp.full((16,), i, jnp.int32))


def _allsum(v):
    lane = lax.iota(jnp.int32, 16)
    for k in (8, 4, 2, 1):
        v = v + _perm(v, lax.bitwise_xor(lane, k))
    return v


def _eproj_tc(ea, We):
    Ep, ed = ea.shape
    BE = 4096

    def body(ea_ref, w_ref, o_ref):
        o_ref[...] = jnp.dot(ea_ref[...], w_ref[...],
                             preferred_element_type=jnp.float32)

    return pl.pallas_call(
        body,
        grid=(Ep // BE,),
        in_specs=[pl.BlockSpec((BE, ed), lambda i: (i, 0)),
                  pl.BlockSpec((ed, HID), lambda i: (0, 0))],
        out_specs=pl.BlockSpec((BE, HID), lambda i: (i, 0)),
        out_shape=jax.ShapeDtypeStruct((Ep, HID), jnp.float32),
    )(ea, We)


def _dense_tc(h_in, acc, bias_prev, la16, Wl, bl, Wr, br, We, att):
    N = h_in.shape[0]
    F = h_in.shape[1] if acc is None else HID
    BN = 2000
    with_epi = acc is not None

    def body(*refs):
        if with_epi:
            (hp_ref, acc_ref, bp_ref, la_ref, wl_ref, bl_ref, wr_ref,
             br_ref, we_ref, att_ref, xl_ref, xre_ref) = refs
            a = acc_ref[...]
            h = (hp_ref[..., :HID] + a[:, :HID]) / (1.0 + a[:, HID:HID + 1] + 1e-16)
            h = jnp.maximum(h + bp_ref[...], 0.0)
        else:
            (hp_ref, la_ref, wl_ref, bl_ref, wr_ref, br_ref, we_ref,
             att_ref, xl_ref, xre_ref) = refs
            h = hp_ref[...]
        la = la_ref[...]
        lattr = la[:, :13] / jnp.maximum(la[:, 13:14], 1.0)
        eL = jnp.dot(lattr, we_ref[...], preferred_element_type=jnp.float32)
        xl = jnp.dot(h, wl_ref[...], preferred_element_type=jnp.float32) + bl_ref[...]
        xr = jnp.dot(h, wr_ref[...], preferred_element_type=jnp.float32) + br_ref[...]
        asf = jnp.sum(_leaky(xl + xr + eL) * att_ref[...], axis=1, keepdims=True)
        xl_ref[...] = jnp.concatenate(
            [xl, jnp.ones((xl.shape[0], 1), jnp.float32),
             jnp.zeros((xl.shape[0], 7), jnp.float32)], axis=1)
        xre_ref[...] = jnp.concatenate(
            [xr, asf, jnp.zeros((xr.shape[0], 15), jnp.float32)], axis=1)

    in_specs = [pl.BlockSpec((BN, h_in.shape[1]), lambda i: (i, 0))]
    args = [h_in]
    if with_epi:
        in_specs += [pl.BlockSpec((BN, 40), lambda i: (i, 0)),
                     pl.BlockSpec((1, HID), lambda i: (0, 0))]
        args += [acc, bias_prev.reshape(1, HID)]
    in_specs += [pl.BlockSpec((BN, 16), lambda i: (i, 0)),
                 pl.BlockSpec((F, HID), lambda i: (0, 0)),
                 pl.BlockSpec((1, HID), lambda i: (0, 0)),
                 pl.BlockSpec((F, HID), lambda i: (0, 0)),
                 pl.BlockSpec((1, HID), lambda i: (0, 0)),
                 pl.BlockSpec((13, HID), lambda i: (0, 0)),
                 pl.BlockSpec((1, HID), lambda i: (0, 0))]
    args += [la16, Wl, bl.reshape(1, HID), Wr, br.reshape(1, HID), We,
             att.reshape(1, HID)]
    return pl.pallas_call(
        body,
        grid=(N // BN,),
        in_specs=in_specs,
        out_specs=[pl.BlockSpec((BN, 40), lambda i: (i, 0)),
                   pl.BlockSpec((BN, 48), lambda i: (i, 0))],
        out_shape=[jax.ShapeDtypeStruct((N, 40), jnp.float32),
                   jax.ShapeDtypeStruct((N, 48), jnp.float32)],
    )(*args)


def _final_tc(xl, acc, bias):
    N = xl.shape[0]
    BN = 2000

    def body(xl_ref, acc_ref, b_ref, o_ref):
        a = acc_ref[...]
        o_ref[...] = (xl_ref[..., :HID] + a[:, :HID]) / (
            1.0 + a[:, HID:HID + 1] + 1e-16) + b_ref[...]

    return pl.pallas_call(
        body,
        grid=(N // BN,),
        in_specs=[pl.BlockSpec((BN, 40), lambda i: (i, 0)),
                  pl.BlockSpec((BN, 40), lambda i: (i, 0)),
                  pl.BlockSpec((1, HID), lambda i: (0, 0))],
        out_specs=pl.BlockSpec((BN, HID), lambda i: (i, 0)),
        out_shape=jax.ShapeDtypeStruct((N, HID), jnp.float32),
    )(xl, acc, bias.reshape(1, HID))


def _sc_mesh():
    return plsc.VectorSubcoreMesh(core_axis_name="c", subcore_axis_name="s",
                                  num_cores=NCORES, num_subcores=NSUB)


def _sc_loop_attr(src, dst, ea1, N, Epad):
    H = N // 2
    Hp = ((H + 128 * NSUB - 1) // (128 * NSUB)) * (128 * NSUB)
    nch = Epad // (NSUB * C)
    rpt = Hp // NSUB
    nz = rpt // 128

    @functools.partial(
        pl.kernel,
        out_type=jax.ShapeDtypeStruct((2 * Hp, 16), jnp.float32),
        mesh=_sc_mesh(),
        compiler_params=pltpu.CompilerParams(use_tc_tiling_on_sc=False),
        scratch_types=[
            pltpu.VMEM((C,), jnp.int32),
            pltpu.VMEM((C,), jnp.int32),
            pltpu.VMEM((C * 13 + 16,), jnp.float32),
            pltpu.VMEM((C, 16), jnp.float32),
            pltpu.VMEM((C // 128, 128), jnp.int32),
            pltpu.VMEM_SHARED((Hp, 16), jnp.float32),
        ],
    )
    def k(src_ref, dst_ref, ea_ref, out_ref, srcb, dstb, eab, rows, idxb, acc):
        cid = lax.axis_index("c")
        sid = lax.axis_index("s")
        base = cid * H
        z16 = jnp.zeros((16,), jnp.float32)

        def zrow(e, _):
            rows[e, pl.ds(0, 16)] = z16
            return 0
        lax.fori_loop(0, C, zrow, 0)
        for z in range(nz):
            pltpu.sync_copy(rows.at[pl.ds(0, 128)],
                            acc.at[pl.ds(sid * rpt + z * 128, 128)])
        plsc.subcore_barrier()

        lane = lax.iota(jnp.int32, 16)

        def chunk(j, _):
            be = (sid * nch + j) * C
            pltpu.sync_copy(src_ref.at[pl.ds(be, C)], srcb)
            pltpu.sync_copy(dst_ref.at[pl.ds(be, C)], dstb)
            pltpu.sync_copy(ea_ref.at[pl.ds(be * 13, C * 13)],
                            eab.at[pl.ds(0, C * 13)])

            def grp(g, _):
                sv16 = srcb[pl.ds(g * 16, 16)]
                dv16 = dstb[pl.ds(g * 16, 16)]
                one = jnp.ones((16,), jnp.float32)
                zero = jnp.zeros((16,), jnp.float32)
                mv16 = (jnp.where(sv16 != dv16, one, zero)
                        * jnp.where(dv16 >= base, one, zero)
                        * jnp.where(dv16 < base + H, one, zero))
                for i in range(16):
                    e = g * 16 + i
                    v = eab[pl.ds(13 * e, 16)]
                    mv = _bcast(mv16, i)
                    r = jnp.where(lane < 13, v * mv,
                                  jnp.where(lane == 13, mv, 0.0))
                    rows[e, pl.ds(0, 16)] = r
                return 0
            lax.fori_loop(0, C // 16, grp, 0)

            for ks in range(C // 128):
                def g16(g, _):
                    dv = dstb[pl.ds(ks * 128 + g * 16, 16)]
                    idxb[ks, pl.ds(g * 16, 16)] = jnp.clip(dv - base, 0, H - 1)
                    return 0
                lax.fori_loop(0, 8, g16, 0)
            for ks in range(C // 128):
                pltpu.sync_copy(rows.at[pl.ds(ks * 128, 128)],
                                acc.at[idxb.at[ks]], add=True)
            return 0
        lax.fori_loop(0, nch, chunk, 0)
        plsc.subcore_barrier()
        pltpu.sync_copy(acc.at[pl.ds(sid * rpt, rpt)],
                        out_ref.at[pl.ds(cid * Hp + sid * rpt, rpt)])

    out = k(src, dst, ea1)
    return jnp.concatenate([out[:H], out[Hp:Hp + H]], axis=0)


def _sc_alpha_pass(src, dst, xl, xre, eproj, att, N, Epad):
    NW = NCORES * NSUB
    nch = Epad // (NW * C)

    @functools.partial(
        pl.kernel,
        out_type=jax.ShapeDtypeStruct((Epad,), jnp.float32),
        mesh=_sc_mesh(),
        compiler_params=pltpu.CompilerParams(use_tc_tiling_on_sc=False),
        scratch_types=[
            pltpu.VMEM((C,), jnp.int32),
            pltpu.VMEM((C,), jnp.int32),
            pltpu.VMEM((C, 40), jnp.float32),
            pltpu.VMEM((C, 48), jnp.float32),
            pltpu.VMEM((C, HID), jnp.float32),
            pltpu.VMEM((HID,), jnp.float32),
            pltpu.VMEM((C,), jnp.float32),
            pltpu.VMEM((C,), jnp.float32),
            pltpu.SemaphoreType.DMA,
            pltpu.SemaphoreType.DMA,
        ],
    )
    def k(src_ref, dst_ref, xl_ref, xre_ref, ep_ref, att_ref, out_ref,
          srcb, dstb, xlb, xrb, epb, attb, ab, exb, sem, sem2):
        cid = lax.axis_index("c")
        sid = lax.axis_index("s")
        w = cid * NSUB + sid
        z16 = jnp.zeros((16,), jnp.float32)
        lane = lax.iota(jnp.int32, 16)
        pltpu.sync_copy(att_ref, attb)

        def chunk(j, _):
            be = (w * nch + j) * C
            c1 = pltpu.async_copy(src_ref.at[pl.ds(be, C)], srcb, sem)
            c2 = pltpu.async_copy(dst_ref.at[pl.ds(be, C)], dstb, sem)
            c3 = pltpu.async_copy(ep_ref.at[pl.ds(be, C)], epb, sem2)
            c1.wait()
            c2.wait()
            cps = []
            for ks in range(C // 128):
                cps.append(pltpu.async_copy(
                    xl_ref.at[srcb.at[pl.ds(ks * 128, 128)]],
                    xlb.at[pl.ds(ks * 128, 128)], sem))
                cps.append(pltpu.async_copy(
                    xre_ref.at[dstb.at[pl.ds(ks * 128, 128)]],
                    xrb.at[pl.ds(ks * 128, 128)], sem2))
            c3.wait()
            for cp in cps:
                cp.wait()

            at0 = attb[pl.ds(0, 16)]
            at1 = attb[pl.ds(16, 16)]

            def grpA(g, _):
                avec = z16
                for i in range(16):
                    e = g * 16 + i
                    v0 = xlb[e, pl.ds(0, 16)] + xrb[e, pl.ds(0, 16)] + epb[e, pl.ds(0, 16)]
                    v1 = xlb[e, pl.ds(16, 16)] + xrb[e, pl.ds(16, 16)] + epb[e, pl.ds(16, 16)]
                    sv = _leaky(v0) * at0 + _leaky(v1) * at1
                    a_i = _allsum(sv) - _bcast(xrb[e, pl.ds(32, 16)], 0)
                    avec = jnp.where(lane == i, a_i, avec)
                ab[pl.ds(g * 16, 16)] = avec
                return 0
            lax.fori_loop(0, C // 16, grpA, 0)

            one = jnp.ones((16,), jnp.float32)
            zero = jnp.zeros((16,), jnp.float32)

            def g16(g, _):
                gg = g * 16
                dv = dstb[pl.ds(gg, 16)]
                sv2 = srcb[pl.ds(gg, 16)]
                av = ab[pl.ds(gg, 16)]
                mf = jnp.where(sv2 != dv, one, zero)
                exb[pl.ds(gg, 16)] = jnp.exp(av) * mf
                return 0
            lax.fori_loop(0, C // 16, g16, 0)
            pltpu.sync_copy(exb, out_ref.at[pl.ds(be, C)])
            return 0
        lax.fori_loop(0, nch, chunk, 0)

    return k(src, dst, xl, xre, eproj, att)


def _sc_scatter_pass(src, dst, ex, xl, N, Epad):
    H = N // 2
    Hp = ((H + 127) // 128) * 128
    C2 = 128
    nch = Epad // (NSUB * C2)
    rpt = Hp // NSUB
    nzf = rpt // 128
    nzr = rpt - nzf * 128

    @functools.partial(
        pl.kernel,
        out_type=jax.ShapeDtypeStruct((2 * Hp, 40), jnp.float32),
        mesh=_sc_mesh(),
        compiler_params=pltpu.CompilerParams(use_tc_tiling_on_sc=False),
        scratch_types=[
            pltpu.VMEM((C2,), jnp.int32),
            pltpu.VMEM((C2,), jnp.int32),
            pltpu.VMEM((C2,), jnp.float32),
            pltpu.VMEM((C2, 40), jnp.float32),
            pltpu.VMEM((C2,), jnp.int32),
            pltpu.VMEM_SHARED((Hp, 40), jnp.float32),
            pltpu.SemaphoreType.DMA,
            pltpu.SemaphoreType.DMA,
        ],
    )
    def k(src_ref, dst_ref, ex_ref, xl_ref, out_ref,
          srcb, dstb, exc, xlb, idxb, acc, sem, sem2):
        cid = lax.axis_index("c")
        sid = lax.axis_index("s")
        base = cid * H
        z16 = jnp.zeros((16,), jnp.float32)
        lane = lax.iota(jnp.int32, 16)

        def zrow(e, _):
            xlb[e, pl.ds(0, 16)] = z16
            xlb[e, pl.ds(16, 16)] = z16
            xlb[e, pl.ds(24, 16)] = z16
            return 0
        lax.fori_loop(0, C2, zrow, 0)
        for z in range(nzf):
            pltpu.sync_copy(xlb.at[pl.ds(0, 128)],
                            acc.at[pl.ds(sid * rpt + z * 128, 128)])
        if nzr:
            pltpu.sync_copy(xlb.at[pl.ds(0, nzr)],
                            acc.at[pl.ds(sid * rpt + nzf * 128, nzr)])
        plsc.subcore_barrier()

        one = jnp.ones((16,), jnp.float32)
        zero = jnp.zeros((16,), jnp.float32)

        def chunk(j, _):
            be = (sid * nch + j) * C2
            pltpu.sync_copy(src_ref.at[pl.ds(be, C2)], srcb)
            g = pltpu.async_copy(xl_ref.at[srcb], xlb, sem)
            c1 = pltpu.async_copy(dst_ref.at[pl.ds(be, C2)], dstb, sem2)
            c2 = pltpu.async_copy(ex_ref.at[pl.ds(be, C2)], exc, sem2)
            c1.wait()
            c2.wait()
            g.wait()

            def g16(g, _):
                gg = g * 16
                dv = dstb[pl.ds(gg, 16)]
                mh = (jnp.where(dv >= base, one, zero)
                      * jnp.where(dv < base + H, one, zero))
                exc[pl.ds(gg, 16)] = exc[pl.ds(gg, 16)] * mh
                idxb[pl.ds(gg, 16)] = jnp.clip(dv - base, 0, H - 1)
                return 0
            lax.fori_loop(0, C2 // 16, g16, 0)

            def grpC(g, _):
                exv16 = exc[pl.ds(g * 16, 16)]
                for i in range(16):
                    e = g * 16 + i
                    exv = _bcast(exv16, i)
                    l0 = xlb[e, pl.ds(0, 16)]
                    l1 = xlb[e, pl.ds(16, 16)]
                    l2 = xlb[e, pl.ds(24, 16)]
                    xlb[e, pl.ds(0, 16)] = l0 * exv
                    xlb[e, pl.ds(16, 16)] = l1 * exv
                    xlb[e, pl.ds(24, 16)] = l2 * exv
                return 0
            lax.fori_loop(0, C2 // 16, grpC, 0)

            pltpu.sync_copy(xlb, acc.at[idxb], add=True)
            return 0
        lax.fori_loop(0, nch, chunk, 0)
        plsc.subcore_barrier()
        pltpu.sync_copy(acc.at[pl.ds(sid * rpt, rpt)],
                        out_ref.at[pl.ds(cid * Hp + sid * rpt, rpt)])

    out = k(src, dst, ex, xl)
    return jnp.concatenate([out[:H], out[Hp:Hp + H]], axis=0)


def kernel(x, edge_index, edge_attr,
           W0l, b0l, W0r, b0r, W0e, att0, bias0,
           W1l, b1l, W1r, b1r, W1e, att1, bias1,
           W2l, b2l, W2r, b2r, W2e, att2, bias2):
    N = x.shape[0]
    E = edge_index.shape[1]
    ed = edge_attr.shape[1]
    chw = NCORES * NSUB * C
    Epad = ((E + chw - 1) // chw) * chw
    pad = Epad - E

    src = jnp.concatenate([edge_index[0], jnp.zeros((pad,), jnp.int32)])
    dst = jnp.concatenate([edge_index[1], jnp.zeros((pad,), jnp.int32)])
    ea = jnp.concatenate([edge_attr, jnp.zeros((pad, ed), jnp.float32)], axis=0)
    ea1 = ea.reshape(-1)

    la16 = _sc_loop_attr(src, dst, ea1, N, Epad)

    params = [
        (W0l, b0l, W0r, b0r, W0e, att0, bias0),
        (W1l, b1l, W1r, b1r, W1e, att1, bias1),
        (W2l, b2l, W2r, b2r, W2e, att2, bias2),
    ]

    h_in = x
    acc = None
    bias_prev = None
    xl = None
    for l in range(3):
        Wl, bl, Wr, br, We, att, bias = params[l]
        eproj = _eproj_tc(ea, We)
        xl, xre = _dense_tc(h_in, acc, bias_prev, la16, Wl, bl, Wr, br, We, att)
        ex = _sc_alpha_pass(src, dst, xl, xre, eproj, att, N, Epad)
        acc = _sc_scatter_pass(src, dst, ex, xl, N, Epad)
        h_in = xl
        bias_prev = bias
    return _final_tc(xl, acc, bias2)
